# baseline (speedup 1.0000x reference)
import functools

import jax
import jax.numpy as jnp
from jax import lax
from jax.experimental import pallas as pl
from jax.experimental.pallas import tpu as pltpu
from jax.experimental.pallas import tpu_sc as plsc

_L = 16
_NC = 2
_NS = 16
_NW = _NC * _NS
_B = 16384
_BPW = _B // _NW
_G = 8
_NG = _BPW // _G
_DIM = 16
_TW = 128


def _masked_idx(idxv, i):
    lane = lax.iota(jnp.int32, _L)
    return jnp.sum(jnp.where(lane == i, idxv, 0))


def _sc_body(uidx_hbm, iidx_hbm, age_hbm, ubas_hbm, ibas_hbm,
             uT_hbm, iT_hbm, cc_hbm, out_hbm,
             uidx_v, iidx_v, age_v, ubas_v, ibas_v,
             bufu_a, bufi_a, bufu_b, bufi_b,
             cc_v, pt_v, out_v, sem_a, sem_b, sem_c):
    wid = lax.axis_index("s") * _NC + lax.axis_index("c")
    base = wid * _BPW

    pltpu.sync_copy(uidx_hbm.at[pl.ds(base, _BPW)], uidx_v.at[pl.ds(0, _BPW)])
    pltpu.sync_copy(iidx_hbm.at[pl.ds(base, _BPW)], iidx_v.at[pl.ds(0, _BPW)])
    pltpu.sync_copy(age_hbm.at[pl.ds(base, _BPW)], age_v)
    pltpu.sync_copy(cc_hbm, cc_v)

    bias_descs = []
    for k in range(_BPW // _L):
        rows = pl.ds(k * _L, _L)
        uv = uidx_v[rows]
        iv = iidx_v[rows]
        bias_descs.append(pltpu.async_copy(ubas_hbm.at[uv], ubas_v.at[rows], sem_c))
        bias_descs.append(pltpu.async_copy(ibas_hbm.at[iv], ibas_v.at[rows], sem_c))
        if len(bias_descs) >= 16:
            for d in bias_descs:
                d.wait()
            bias_descs = []
    for d in bias_descs:
        d.wait()

    lane = lax.iota(jnp.int32, _L)
    tpos = lane * _L

    def fire(g, bufu, bufi, sem):
        uv = uidx_v[pl.ds(g * _G, _L)]
        iv = iidx_v[pl.ds(g * _G, _L)]
        for i in range(_G):
            ru = _masked_idx(uv, i)
            ri = _masked_idx(iv, i)
            su = pl.multiple_of((ru // _TW) * _TW, _TW)
            si = pl.multiple_of((ri // _TW) * _TW, _TW)
            pltpu.async_copy(uT_hbm.at[:, pl.ds(su, _TW)], bufu.at[i], sem)
            pltpu.async_copy(iT_hbm.at[:, pl.ds(si, _TW)], bufi.at[i], sem)

    def drain(bufu, bufi, sem):
        for i in range(_G):
            pltpu.make_async_copy(uT_hbm.at[:, pl.ds(0, _TW)], bufu.at[i], sem).wait()
            pltpu.make_async_copy(iT_hbm.at[:, pl.ds(0, _TW)], bufi.at[i], sem).wait()

    def process(g, bufu, bufi):
        uv = uidx_v[pl.ds(g * _G, _L)]
        iv = iidx_v[pl.ds(g * _G, _L)]
        m0 = (g % 2) * _G
        for i in range(_G):
            ru = _masked_idx(uv, i)
            ri = _masked_idx(iv, i)
            roffu = jnp.full((_L,), ru % _TW, jnp.int32)
            roffi = jnp.full((_L,), ri % _TW, jnp.int32)
            item = jnp.full((_L,), i, jnp.int32)
            u = plsc.load_gather(bufu, [item, lane, roffu])
            v = plsc.load_gather(bufi, [item, lane, roffi])
            plsc.store_scatter(pt_v, [tpos + (m0 + i)], u * v)

    fire(0, bufu_a, bufi_a, sem_a)

    def pair(p, carry):
        g0 = 2 * p
        fire(g0 + 1, bufu_b, bufi_b, sem_b)
        drain(bufu_a, bufi_a, sem_a)
        process(g0, bufu_a, bufi_a)

        @pl.when(p < _NG // 2 - 1)
        def _():
            fire(g0 + 2, bufu_a, bufi_a, sem_a)

        drain(bufu_b, bufi_b, sem_b)
        process(g0 + 1, bufu_b, bufi_b)

        acc = pt_v[pl.ds(0, _L)]
        for j in range(1, _DIM):
            acc = acc + pt_v[pl.ds(j * _L, _L)]
        out_v[pl.ds(p * _L, _L)] = acc
        return carry

    lax.fori_loop(0, _NG // 2, pair, 0)

    c0 = cc_v[pl.ds(0, _L)]
    c1 = cc_v[pl.ds(_L, _L)]

    def finish(k, carry):
        rows = pl.ds(k * _L, _L)
        out_v[rows] = (out_v[rows] + ubas_v[rows] + ibas_v[rows]
                       + c0 + c1 * age_v[rows])
        return carry

    lax.fori_loop(0, _BPW // _L, finish, 0)
    pltpu.sync_copy(out_v, out_hbm.at[pl.ds(base, _BPW)])


@functools.partial(
    pl.kernel,
    out_type=jax.ShapeDtypeStruct((_B,), jnp.float32),
    mesh=plsc.VectorSubcoreMesh(core_axis_name="c", subcore_axis_name="s"),
    compiler_params=pltpu.CompilerParams(
        needs_layout_passes=False, use_tc_tiling_on_sc=True,
        skip_device_barrier=True,
        disable_bounds_checks=True, disable_semaphore_checks=True),
    scratch_types=[
        pltpu.VMEM((_BPW + _L,), jnp.int32),
        pltpu.VMEM((_BPW + _L,), jnp.int32),
        pltpu.VMEM((_BPW,), jnp.float32),
        pltpu.VMEM((_BPW,), jnp.float32),
        pltpu.VMEM((_BPW,), jnp.float32),
        pltpu.VMEM((_G, _DIM, _TW), jnp.float32),
        pltpu.VMEM((_G, _DIM, _TW), jnp.float32),
        pltpu.VMEM((_G, _DIM, _TW), jnp.float32),
        pltpu.VMEM((_G, _DIM, _TW), jnp.float32),
        pltpu.VMEM((2 * _L,), jnp.float32),
        pltpu.VMEM((_L * _DIM,), jnp.float32),
        pltpu.VMEM((_BPW,), jnp.float32),
        pltpu.SemaphoreType.DMA,
        pltpu.SemaphoreType.DMA,
        pltpu.SemaphoreType.DMA,
    ],
)
def _sc_score(uidx_hbm, iidx_hbm, age_hbm, ubas_hbm, ibas_hbm,
              uT_hbm, iT_hbm, cc_hbm, out_hbm,
              uidx_v, iidx_v, age_v, ubas_v, ibas_v,
              bufu_a, bufi_a, bufu_b, bufi_b,
              cc_v, pt_v, out_v, sem_a, sem_b, sem_c):
    _sc_body(uidx_hbm, iidx_hbm, age_hbm, ubas_hbm, ibas_hbm,
             uT_hbm, iT_hbm, cc_hbm, out_hbm,
             uidx_v, iidx_v, age_v, ubas_v, ibas_v,
             bufu_a, bufi_a, bufu_b, bufi_b,
             cc_v, pt_v, out_v, sem_a, sem_b, sem_c)


def kernel(user_idx, item_idx, age, glob_bas, user_bas_w, item_bas_w,
           user_vec_w, item_vec_w, age1_w, age1_b, age2_w, age2_b):
    c1 = age1_w[0, 0] * age2_w[0, 0]
    c0 = glob_bas[0, 0] + age1_b[0] * age2_w[0, 0] + age2_b[0]
    cc = jnp.concatenate([
        jnp.full((_L,), c0, jnp.float32),
        jnp.full((_L,), c1, jnp.float32),
    ])
    return _sc_score(user_idx.astype(jnp.int32), item_idx.astype(jnp.int32),
                     age.astype(jnp.float32),
                     user_bas_w.reshape(-1), item_bas_w.reshape(-1),
                     user_vec_w.T, item_vec_w.T, cc)


# speedup vs baseline: 1.0085x; 1.0085x over previous
import functools

import jax
import jax.numpy as jnp
from jax import lax
from jax.experimental import pallas as pl
from jax.experimental.pallas import tpu as pltpu
from jax.experimental.pallas import tpu_sc as plsc

_L = 16
_NC = 2
_NS = 16
_NW = _NC * _NS
_B = 16384
_BPW = _B // _NW
_G = 8
_NG = _BPW // _G
_DIM = 16
_TW = 128


def _masked_idx(idxv, i):
    lane = lax.iota(jnp.int32, _L)
    return jnp.sum(jnp.where(lane == i, idxv, 0))


def _sc_body(uidx_hbm, iidx_hbm, age_hbm, ubas_hbm, ibas_hbm,
             uT_hbm, iT_hbm, cc_hbm, out_hbm,
             uidx_v, iidx_v, age_v, ubas_v, ibas_v,
             bufu_a, bufi_a, bufu_b, bufi_b,
             cc_v, pt_v, out_v, sem_a, sem_b, sem_c):
    wid = lax.axis_index("s") * _NC + lax.axis_index("c")
    base = wid * _BPW

    pltpu.sync_copy(uidx_hbm.at[pl.ds(base, _BPW)], uidx_v.at[pl.ds(0, _BPW)])
    pltpu.sync_copy(iidx_hbm.at[pl.ds(base, _BPW)], iidx_v.at[pl.ds(0, _BPW)])
    pltpu.sync_copy(age_hbm.at[pl.ds(base, _BPW)], age_v)
    pltpu.sync_copy(cc_hbm, cc_v)

    for k in range(_BPW // _L):
        rows = pl.ds(k * _L, _L)
        uv = uidx_v[rows]
        iv = iidx_v[rows]
        pltpu.async_copy(ubas_hbm.at[uv], ubas_v.at[rows], sem_c)
        pltpu.async_copy(ibas_hbm.at[iv], ibas_v.at[rows], sem_c)

    lane = lax.iota(jnp.int32, _L)
    tpos = lane * _L

    def fire(g, bufu, bufi, sem):
        uv = uidx_v[pl.ds(g * _G, _L)]
        iv = iidx_v[pl.ds(g * _G, _L)]
        for i in range(_G):
            ru = _masked_idx(uv, i)
            ri = _masked_idx(iv, i)
            su = pl.multiple_of((ru // _TW) * _TW, _TW)
            si = pl.multiple_of((ri // _TW) * _TW, _TW)
            pltpu.async_copy(uT_hbm.at[:, pl.ds(su, _TW)], bufu.at[i], sem)
            pltpu.async_copy(iT_hbm.at[:, pl.ds(si, _TW)], bufi.at[i], sem)

    def drain(bufu, bufi, sem):
        for i in range(_G):
            pltpu.make_async_copy(uT_hbm.at[:, pl.ds(0, _TW)], bufu.at[i], sem).wait()
            pltpu.make_async_copy(iT_hbm.at[:, pl.ds(0, _TW)], bufi.at[i], sem).wait()

    def process(g, bufu, bufi):
        uv = uidx_v[pl.ds(g * _G, _L)]
        iv = iidx_v[pl.ds(g * _G, _L)]
        m0 = (g % 2) * _G
        for i in range(_G):
            ru = _masked_idx(uv, i)
            ri = _masked_idx(iv, i)
            roffu = jnp.full((_L,), ru % _TW, jnp.int32)
            roffi = jnp.full((_L,), ri % _TW, jnp.int32)
            item = jnp.full((_L,), i, jnp.int32)
            u = plsc.load_gather(bufu, [item, lane, roffu])
            v = plsc.load_gather(bufi, [item, lane, roffi])
            plsc.store_scatter(pt_v, [tpos + (m0 + i)], u * v)

    fire(0, bufu_a, bufi_a, sem_a)

    def pair(p, carry):
        g0 = 2 * p
        fire(g0 + 1, bufu_b, bufi_b, sem_b)
        drain(bufu_a, bufi_a, sem_a)
        process(g0, bufu_a, bufi_a)

        @pl.when(p < _NG // 2 - 1)
        def _():
            fire(g0 + 2, bufu_a, bufi_a, sem_a)

        drain(bufu_b, bufi_b, sem_b)
        process(g0 + 1, bufu_b, bufi_b)

        acc = pt_v[pl.ds(0, _L)]
        for j in range(1, _DIM):
            acc = acc + pt_v[pl.ds(j * _L, _L)]
        out_v[pl.ds(p * _L, _L)] = acc
        return carry

    lax.fori_loop(0, _NG // 2, pair, 0)

    for k in range(_BPW // _L):
        rows = pl.ds(k * _L, _L)
        pltpu.make_async_copy(ubas_hbm.at[uidx_v[rows]], ubas_v.at[rows], sem_c).wait()
        pltpu.make_async_copy(ibas_hbm.at[iidx_v[rows]], ibas_v.at[rows], sem_c).wait()
    c0 = cc_v[pl.ds(0, _L)]
    c1 = cc_v[pl.ds(_L, _L)]

    def finish(k, carry):
        rows = pl.ds(k * _L, _L)
        out_v[rows] = (out_v[rows] + ubas_v[rows] + ibas_v[rows]
                       + c0 + c1 * age_v[rows])
        return carry

    lax.fori_loop(0, _BPW // _L, finish, 0)
    pltpu.sync_copy(out_v, out_hbm.at[pl.ds(base, _BPW)])


@functools.partial(
    pl.kernel,
    out_type=jax.ShapeDtypeStruct((_B,), jnp.float32),
    mesh=plsc.VectorSubcoreMesh(core_axis_name="c", subcore_axis_name="s"),
    compiler_params=pltpu.CompilerParams(
        needs_layout_passes=False, use_tc_tiling_on_sc=True,
        skip_device_barrier=True,
        disable_bounds_checks=True, disable_semaphore_checks=True),
    scratch_types=[
        pltpu.VMEM((_BPW + _L,), jnp.int32),
        pltpu.VMEM((_BPW + _L,), jnp.int32),
        pltpu.VMEM((_BPW,), jnp.float32),
        pltpu.VMEM((_BPW,), jnp.float32),
        pltpu.VMEM((_BPW,), jnp.float32),
        pltpu.VMEM((_G, _DIM, _TW), jnp.float32),
        pltpu.VMEM((_G, _DIM, _TW), jnp.float32),
        pltpu.VMEM((_G, _DIM, _TW), jnp.float32),
        pltpu.VMEM((_G, _DIM, _TW), jnp.float32),
        pltpu.VMEM((2 * _L,), jnp.float32),
        pltpu.VMEM((_L * _DIM,), jnp.float32),
        pltpu.VMEM((_BPW,), jnp.float32),
        pltpu.SemaphoreType.DMA,
        pltpu.SemaphoreType.DMA,
        pltpu.SemaphoreType.DMA,
    ],
)
def _sc_score(uidx_hbm, iidx_hbm, age_hbm, ubas_hbm, ibas_hbm,
              uT_hbm, iT_hbm, cc_hbm, out_hbm,
              uidx_v, iidx_v, age_v, ubas_v, ibas_v,
              bufu_a, bufi_a, bufu_b, bufi_b,
              cc_v, pt_v, out_v, sem_a, sem_b, sem_c):
    _sc_body(uidx_hbm, iidx_hbm, age_hbm, ubas_hbm, ibas_hbm,
             uT_hbm, iT_hbm, cc_hbm, out_hbm,
             uidx_v, iidx_v, age_v, ubas_v, ibas_v,
             bufu_a, bufi_a, bufu_b, bufi_b,
             cc_v, pt_v, out_v, sem_a, sem_b, sem_c)


def kernel(user_idx, item_idx, age, glob_bas, user_bas_w, item_bas_w,
           user_vec_w, item_vec_w, age1_w, age1_b, age2_w, age2_b):
    c1 = age1_w[0, 0] * age2_w[0, 0]
    c0 = glob_bas[0, 0] + age1_b[0] * age2_w[0, 0] + age2_b[0]
    cc = jnp.concatenate([
        jnp.full((_L,), c0, jnp.float32),
        jnp.full((_L,), c1, jnp.float32),
    ])
    return _sc_score(user_idx.astype(jnp.int32), item_idx.astype(jnp.int32),
                     age.astype(jnp.float32),
                     user_bas_w.reshape(-1), item_bas_w.reshape(-1),
                     user_vec_w.T, item_vec_w.T, cc)


# speedup vs baseline: 1.6549x; 1.6410x over previous
import functools

import jax
import jax.numpy as jnp
from jax import lax
from jax.experimental import pallas as pl
from jax.experimental.pallas import tpu as pltpu
from jax.experimental.pallas import tpu_sc as plsc

_L = 16
_NC = 2
_NS = 16
_NW = _NC * _NS
_B = 16384
_BPW = _B // _NW
_G = 8
_NG = _BPW // _G
_DIM = 16
_TW = 128


def _masked_idx(idxv, i):
    lane = lax.iota(jnp.int32, _L)
    return jnp.sum(jnp.where(lane == i, idxv, 0))


def _sc_body(uidx_hbm, iidx_hbm, age_hbm, ubas_hbm, ibas_hbm,
             uT_hbm, iT_hbm, cc_hbm, out_hbm,
             uidx_v, iidx_v, age_v, ubas_v, ibas_v,
             bufu_a, bufi_a, bufu_b, bufi_b,
             cc_v, pt_v, out_v, sem_a, sem_b, sem_c):
    wid = lax.axis_index("s") * _NC + lax.axis_index("c")
    base = wid * _BPW

    pltpu.sync_copy(uidx_hbm.at[pl.ds(base, _BPW)], uidx_v.at[pl.ds(0, _BPW)])
    pltpu.sync_copy(iidx_hbm.at[pl.ds(base, _BPW)], iidx_v.at[pl.ds(0, _BPW)])
    pltpu.sync_copy(age_hbm.at[pl.ds(base, _BPW)], age_v)
    pltpu.sync_copy(cc_hbm, cc_v)

    for k in range(_BPW // _L):
        rows = pl.ds(k * _L, _L)
        uv = uidx_v[rows]
        iv = iidx_v[rows]
        pltpu.async_copy(ubas_hbm.at[0].at[uv], ubas_v.at[rows], sem_c)
        pltpu.async_copy(ibas_hbm.at[0].at[iv], ibas_v.at[rows], sem_c)

    lane = lax.iota(jnp.int32, _L)
    tpos = lane * _L

    def fire(g, bufu, bufi, sem):
        uv = uidx_v[pl.ds(g * _G, _L)]
        iv = iidx_v[pl.ds(g * _G, _L)]
        for i in range(_G):
            ru = _masked_idx(uv, i)
            ri = _masked_idx(iv, i)
            su = pl.multiple_of((ru // _TW) * _TW, _TW)
            si = pl.multiple_of((ri // _TW) * _TW, _TW)
            pltpu.async_copy(uT_hbm.at[:, pl.ds(su, _TW)], bufu.at[i], sem)
            pltpu.async_copy(iT_hbm.at[:, pl.ds(si, _TW)], bufi.at[i], sem)

    def drain(bufu, bufi, sem):
        for i in range(_G):
            pltpu.make_async_copy(uT_hbm.at[:, pl.ds(0, _TW)], bufu.at[i], sem).wait()
            pltpu.make_async_copy(iT_hbm.at[:, pl.ds(0, _TW)], bufi.at[i], sem).wait()

    def process(g, bufu, bufi):
        uv = uidx_v[pl.ds(g * _G, _L)]
        iv = iidx_v[pl.ds(g * _G, _L)]
        m0 = (g % 2) * _G
        for i in range(_G):
            ru = _masked_idx(uv, i)
            ri = _masked_idx(iv, i)
            roffu = jnp.full((_L,), ru % _TW, jnp.int32)
            roffi = jnp.full((_L,), ri % _TW, jnp.int32)
            item = jnp.full((_L,), i, jnp.int32)
            u = plsc.load_gather(bufu, [item, lane, roffu])
            v = plsc.load_gather(bufi, [item, lane, roffi])
            plsc.store_scatter(pt_v, [tpos + (m0 + i)], u * v)

    fire(0, bufu_a, bufi_a, sem_a)

    def pair(p, carry):
        g0 = 2 * p
        fire(g0 + 1, bufu_b, bufi_b, sem_b)
        drain(bufu_a, bufi_a, sem_a)
        process(g0, bufu_a, bufi_a)

        @pl.when(p < _NG // 2 - 1)
        def _():
            fire(g0 + 2, bufu_a, bufi_a, sem_a)

        drain(bufu_b, bufi_b, sem_b)
        process(g0 + 1, bufu_b, bufi_b)

        acc = pt_v[pl.ds(0, _L)]
        for j in range(1, _DIM):
            acc = acc + pt_v[pl.ds(j * _L, _L)]
        out_v[pl.ds(p * _L, _L)] = acc
        return carry

    lax.fori_loop(0, _NG // 2, pair, 0)

    for k in range(_BPW // _L):
        rows = pl.ds(k * _L, _L)
        pltpu.make_async_copy(ubas_hbm.at[0].at[uidx_v[rows]], ubas_v.at[rows], sem_c).wait()
        pltpu.make_async_copy(ibas_hbm.at[0].at[iidx_v[rows]], ibas_v.at[rows], sem_c).wait()
    c0 = cc_v[pl.ds(0, _L)]
    c1 = cc_v[pl.ds(_L, _L)]

    def finish(k, carry):
        rows = pl.ds(k * _L, _L)
        out_v[rows] = (out_v[rows] + ubas_v[rows] + ibas_v[rows]
                       + c0 + c1 * age_v[rows])
        return carry

    lax.fori_loop(0, _BPW // _L, finish, 0)
    pltpu.sync_copy(out_v, out_hbm.at[pl.ds(base, _BPW)])


@functools.partial(
    pl.kernel,
    out_type=jax.ShapeDtypeStruct((_B,), jnp.float32),
    mesh=plsc.VectorSubcoreMesh(core_axis_name="c", subcore_axis_name="s"),
    compiler_params=pltpu.CompilerParams(
        needs_layout_passes=False, use_tc_tiling_on_sc=True,
        skip_device_barrier=True,
        disable_bounds_checks=True, disable_semaphore_checks=True),
    scratch_types=[
        pltpu.VMEM((_BPW + _L,), jnp.int32),
        pltpu.VMEM((_BPW + _L,), jnp.int32),
        pltpu.VMEM((_BPW,), jnp.float32),
        pltpu.VMEM((_BPW,), jnp.float32),
        pltpu.VMEM((_BPW,), jnp.float32),
        pltpu.VMEM((_G, _DIM, _TW), jnp.float32),
        pltpu.VMEM((_G, _DIM, _TW), jnp.float32),
        pltpu.VMEM((_G, _DIM, _TW), jnp.float32),
        pltpu.VMEM((_G, _DIM, _TW), jnp.float32),
        pltpu.VMEM((2 * _L,), jnp.float32),
        pltpu.VMEM((_L * _DIM,), jnp.float32),
        pltpu.VMEM((_BPW,), jnp.float32),
        pltpu.SemaphoreType.DMA,
        pltpu.SemaphoreType.DMA,
        pltpu.SemaphoreType.DMA,
    ],
)
def _sc_score(uidx_hbm, iidx_hbm, age_hbm, ubas_hbm, ibas_hbm,
              uT_hbm, iT_hbm, cc_hbm, out_hbm,
              uidx_v, iidx_v, age_v, ubas_v, ibas_v,
              bufu_a, bufi_a, bufu_b, bufi_b,
              cc_v, pt_v, out_v, sem_a, sem_b, sem_c):
    _sc_body(uidx_hbm, iidx_hbm, age_hbm, ubas_hbm, ibas_hbm,
             uT_hbm, iT_hbm, cc_hbm, out_hbm,
             uidx_v, iidx_v, age_v, ubas_v, ibas_v,
             bufu_a, bufi_a, bufu_b, bufi_b,
             cc_v, pt_v, out_v, sem_a, sem_b, sem_c)


def kernel(user_idx, item_idx, age, glob_bas, user_bas_w, item_bas_w,
           user_vec_w, item_vec_w, age1_w, age1_b, age2_w, age2_b):
    c1 = age1_w[0, 0] * age2_w[0, 0]
    c0 = glob_bas[0, 0] + age1_b[0] * age2_w[0, 0] + age2_b[0]
    cc = jnp.concatenate([
        jnp.full((_L,), c0, jnp.float32),
        jnp.full((_L,), c1, jnp.float32),
    ])
    return _sc_score(user_idx.astype(jnp.int32), item_idx.astype(jnp.int32),
                     age.astype(jnp.float32),
                     user_bas_w.T, item_bas_w.T,
                     user_vec_w.T, item_vec_w.T, cc)
